# baseline (speedup 1.0000x reference)
import functools

import jax
import jax.numpy as jnp
from jax.experimental import pallas as pl
from jax.experimental.pallas import tpu as pltpu

NUM_CLASSES_K = 100000
SMOOTHING_K = 0.1
CONFIDENCE_K = 1.0 - SMOOTHING_K
EPS_K = SMOOTHING_K / (NUM_CLASSES_K - 1)

ROW_BLOCK = 16


def _block_kernel(pred_ref, tgt_ref, out_ref, *, num_cols, total_rows):
    m = jnp.max(pred_ref[...], axis=1, keepdims=True)
    s = jnp.sum(jnp.exp(pred_ref[...] - m), axis=1, keepdims=True)
    t = jnp.sum(pred_ref[...], axis=1, keepdims=True)

    col = jax.lax.broadcasted_iota(jnp.int32, pred_ref.shape, 1)
    g = jnp.sum(jnp.where(col == tgt_ref[...], pred_ref[...], 0.0),
                axis=1, keepdims=True)

    lse = m + jnp.log(s)
    row_loss = (-EPS_K * (t - num_cols * lse)
                - (CONFIDENCE_K - EPS_K) * (g - lse))
    out_ref[...] = jnp.sum(row_loss).reshape(1, 1, 1) / total_rows


def _sum_kernel(parts_ref, out_ref):
    out_ref[...] = jnp.sum(parts_ref[...]).reshape(1, 1)


def kernel(pred, target):
    rows, num_cols = pred.shape
    num_blocks = rows // ROW_BLOCK
    tgt2d = target.astype(jnp.int32).reshape(rows, 1)

    parts = pl.pallas_call(
        functools.partial(_block_kernel, num_cols=num_cols,
                          total_rows=rows),
        grid=(num_blocks,),
        in_specs=[
            pl.BlockSpec((ROW_BLOCK, num_cols), lambda i: (i, 0)),
            pl.BlockSpec((ROW_BLOCK, 1), lambda i: (i, 0)),
        ],
        out_specs=pl.BlockSpec((1, 1, 1), lambda i: (i, 0, 0)),
        out_shape=jax.ShapeDtypeStruct((num_blocks, 1, 1), jnp.float32),
        compiler_params=pltpu.CompilerParams(
            dimension_semantics=("parallel",)),
    )(pred, tgt2d)

    out = pl.pallas_call(
        _sum_kernel,
        in_specs=[pl.BlockSpec((num_blocks, 1, 1), lambda: (0, 0, 0))],
        out_specs=pl.BlockSpec((1, 1), lambda: (0, 0)),
        out_shape=jax.ShapeDtypeStruct((1, 1), jnp.float32),
    )(parts)
    return out[0, 0]


# speedup vs baseline: 3.3205x; 3.3205x over previous
import functools

import jax
import jax.numpy as jnp
from jax.experimental import pallas as pl
from jax.experimental.pallas import tpu as pltpu

NUM_CLASSES_K = 100000
SMOOTHING_K = 0.1
CONFIDENCE_K = 1.0 - SMOOTHING_K
EPS_K = SMOOTHING_K / (NUM_CLASSES_K - 1)

CLASS_CHUNK = 2048


def _accumulate(x_ref, row0, tgt, m_ref, s_ref, t_ref, g_ref, *, num_classes,
                masked):
    row = jax.lax.broadcasted_iota(jnp.int32, x_ref.shape, 0) + row0
    if masked:
        valid = row < num_classes
        x_max_in = jnp.where(valid, x_ref[...], -jnp.inf)
        x_sum_in = jnp.where(valid, x_ref[...], 0.0)
    else:
        x_max_in = x_ref[...]
        x_sum_in = x_ref[...]

    m_old = m_ref[...]
    chunk_max = jnp.max(x_max_in, axis=0, keepdims=True)
    m_new = jnp.maximum(m_old, chunk_max)
    s_ref[...] = s_ref[...] * jnp.exp(m_old - m_new) + jnp.sum(
        jnp.exp(x_max_in - m_new), axis=0, keepdims=True)
    m_ref[...] = m_new

    t_ref[...] = t_ref[...] + jnp.sum(x_sum_in, axis=0, keepdims=True)

    g_ref[...] = g_ref[...] + jnp.sum(
        jnp.where(row == tgt, x_ref[...], 0.0), axis=0, keepdims=True)


def _loss_kernel(xt_ref, tgt_ref, out_ref, m_ref, s_ref, t_ref, g_ref,
                 *, num_chunks, num_classes, total_rows):
    j = pl.program_id(0)

    @pl.when(j == 0)
    def _init():
        m_ref[...] = jnp.full_like(m_ref, -jnp.inf)
        s_ref[...] = jnp.zeros_like(s_ref)
        t_ref[...] = jnp.zeros_like(t_ref)
        g_ref[...] = jnp.zeros_like(g_ref)

    tgt = tgt_ref[...]
    row0 = j * CLASS_CHUNK

    @pl.when(j < num_chunks - 1)
    def _full():
        _accumulate(xt_ref, row0, tgt, m_ref, s_ref, t_ref, g_ref,
                    num_classes=num_classes, masked=False)

    @pl.when(j == num_chunks - 1)
    def _last():
        _accumulate(xt_ref, row0, tgt, m_ref, s_ref, t_ref, g_ref,
                    num_classes=num_classes,
                    masked=(num_classes % CLASS_CHUNK != 0))
        lse = m_ref[...] + jnp.log(s_ref[...])
        row_loss = (-EPS_K * (t_ref[...] - num_classes * lse)
                    - (CONFIDENCE_K - EPS_K) * (g_ref[...] - lse))
        out_ref[...] = jnp.sum(row_loss).reshape(1, 1) / total_rows


def kernel(pred, target):
    rows, num_classes = pred.shape
    xt = pred.T
    num_chunks = pl.cdiv(num_classes, CLASS_CHUNK)
    tgt2d = target.astype(jnp.int32).reshape(1, rows)

    out = pl.pallas_call(
        functools.partial(_loss_kernel, num_chunks=num_chunks,
                          num_classes=num_classes, total_rows=rows),
        grid=(num_chunks,),
        in_specs=[
            pl.BlockSpec((CLASS_CHUNK, rows), lambda j: (j, 0)),
            pl.BlockSpec((1, rows), lambda j: (0, 0)),
        ],
        out_specs=pl.BlockSpec((1, 1), lambda j: (0, 0)),
        out_shape=jax.ShapeDtypeStruct((1, 1), jnp.float32),
        scratch_shapes=[
            pltpu.VMEM((1, rows), jnp.float32),
            pltpu.VMEM((1, rows), jnp.float32),
            pltpu.VMEM((1, rows), jnp.float32),
            pltpu.VMEM((1, rows), jnp.float32),
        ],
    )(xt, tgt2d)
    return out[0, 0]


# speedup vs baseline: 3.7491x; 1.1291x over previous
import functools

import jax
import jax.numpy as jnp
from jax.experimental import pallas as pl
from jax.experimental.pallas import tpu as pltpu

NUM_CLASSES_K = 100000
SMOOTHING_K = 0.1
CONFIDENCE_K = 1.0 - SMOOTHING_K
EPS_K = SMOOTHING_K / (NUM_CLASSES_K - 1)

CLASS_CHUNK = 2048


def _colsum(x):
    ones = jnp.ones((1, x.shape[0]), jnp.float32)
    return jax.lax.dot_general(ones, x, (((1,), (0,)), ((), ())),
                               preferred_element_type=jnp.float32)


def _accumulate(x_ref, row0, tgt, m_ref, s_ref, t_ref, g_ref, *, num_classes,
                masked):
    row = jax.lax.broadcasted_iota(jnp.int32, x_ref.shape, 0) + row0
    if masked:
        valid = row < num_classes
        x_max_in = jnp.where(valid, x_ref[...], -jnp.inf)
        x_sum_in = jnp.where(valid, x_ref[...], 0.0)
    else:
        x_max_in = x_ref[...]
        x_sum_in = x_ref[...]

    m_old = m_ref[...]
    chunk_max = jnp.max(x_max_in, axis=0, keepdims=True)
    m_new = jnp.maximum(m_old, chunk_max)
    s_ref[...] = s_ref[...] * jnp.exp(m_old - m_new) + _colsum(
        jnp.exp(x_max_in - m_new))
    m_ref[...] = m_new

    t_ref[...] = t_ref[...] + _colsum(x_sum_in)

    g_ref[...] = g_ref[...] + _colsum(
        jnp.where(row == tgt, x_ref[...], 0.0))


def _loss_kernel(xt_ref, tgt_ref, out_ref, m_ref, s_ref, t_ref, g_ref,
                 *, num_chunks, num_classes, total_rows):
    j = pl.program_id(0)

    @pl.when(j == 0)
    def _init():
        m_ref[...] = jnp.full_like(m_ref, -jnp.inf)
        s_ref[...] = jnp.zeros_like(s_ref)
        t_ref[...] = jnp.zeros_like(t_ref)
        g_ref[...] = jnp.zeros_like(g_ref)

    tgt = tgt_ref[...]
    row0 = j * CLASS_CHUNK

    @pl.when(j < num_chunks - 1)
    def _full():
        _accumulate(xt_ref, row0, tgt, m_ref, s_ref, t_ref, g_ref,
                    num_classes=num_classes, masked=False)

    @pl.when(j == num_chunks - 1)
    def _last():
        _accumulate(xt_ref, row0, tgt, m_ref, s_ref, t_ref, g_ref,
                    num_classes=num_classes,
                    masked=(num_classes % CLASS_CHUNK != 0))
        lse = m_ref[...] + jnp.log(s_ref[...])
        row_loss = (-EPS_K * (t_ref[...] - num_classes * lse)
                    - (CONFIDENCE_K - EPS_K) * (g_ref[...] - lse))
        out_ref[...] = jnp.sum(row_loss).reshape(1, 1) / total_rows


def kernel(pred, target):
    rows, num_classes = pred.shape
    xt = pred.T
    num_chunks = pl.cdiv(num_classes, CLASS_CHUNK)
    tgt2d = target.astype(jnp.int32).reshape(1, rows)

    out = pl.pallas_call(
        functools.partial(_loss_kernel, num_chunks=num_chunks,
                          num_classes=num_classes, total_rows=rows),
        grid=(num_chunks,),
        in_specs=[
            pl.BlockSpec((CLASS_CHUNK, rows), lambda j: (j, 0)),
            pl.BlockSpec((1, rows), lambda j: (0, 0)),
        ],
        out_specs=pl.BlockSpec((1, 1), lambda j: (0, 0)),
        out_shape=jax.ShapeDtypeStruct((1, 1), jnp.float32),
        scratch_shapes=[
            pltpu.VMEM((1, rows), jnp.float32),
            pltpu.VMEM((1, rows), jnp.float32),
            pltpu.VMEM((1, rows), jnp.float32),
            pltpu.VMEM((1, rows), jnp.float32),
        ],
    )(xt, tgt2d)
    return out[0, 0]


# speedup vs baseline: 4.0204x; 1.0724x over previous
import functools

import jax
import jax.numpy as jnp
from jax.experimental import pallas as pl
from jax.experimental.pallas import tpu as pltpu

NUM_CLASSES_K = 100000
SMOOTHING_K = 0.1
CONFIDENCE_K = 1.0 - SMOOTHING_K
EPS_K = SMOOTHING_K / (NUM_CLASSES_K - 1)

CLASS_CHUNK = 4096


def _colsum(x):
    ones = jnp.ones((1, x.shape[0]), jnp.float32)
    return jax.lax.dot_general(ones, x, (((1,), (0,)), ((), ())),
                               preferred_element_type=jnp.float32)


def _accumulate(x_ref, row0, tgt, m_ref, s_ref, t_ref, g_ref, *, num_classes,
                masked):
    row = jax.lax.broadcasted_iota(jnp.int32, x_ref.shape, 0) + row0
    if masked:
        valid = row < num_classes
        x_max_in = jnp.where(valid, x_ref[...], -jnp.inf)
        x_sum_in = jnp.where(valid, x_ref[...], 0.0)
    else:
        x_max_in = x_ref[...]
        x_sum_in = x_ref[...]

    m_old = m_ref[...]
    chunk_max = jnp.max(x_max_in, axis=0, keepdims=True)
    m_new = jnp.maximum(m_old, chunk_max)
    s_ref[...] = s_ref[...] * jnp.exp(m_old - m_new) + _colsum(
        jnp.exp(x_max_in - m_new))
    m_ref[...] = m_new

    t_ref[...] = t_ref[...] + _colsum(x_sum_in)

    g_ref[...] = g_ref[...] + _colsum(
        jnp.where(row == tgt, x_ref[...], 0.0))


def _loss_kernel(xt_ref, tgt_ref, out_ref, m_ref, s_ref, t_ref, g_ref,
                 *, num_chunks, num_classes, total_rows):
    j = pl.program_id(0)

    @pl.when(j == 0)
    def _init():
        m_ref[...] = jnp.full_like(m_ref, -jnp.inf)
        s_ref[...] = jnp.zeros_like(s_ref)
        t_ref[...] = jnp.zeros_like(t_ref)
        g_ref[...] = jnp.zeros_like(g_ref)

    tgt = tgt_ref[...]
    row0 = j * CLASS_CHUNK

    @pl.when(j < num_chunks - 1)
    def _full():
        _accumulate(xt_ref, row0, tgt, m_ref, s_ref, t_ref, g_ref,
                    num_classes=num_classes, masked=False)

    @pl.when(j == num_chunks - 1)
    def _last():
        _accumulate(xt_ref, row0, tgt, m_ref, s_ref, t_ref, g_ref,
                    num_classes=num_classes,
                    masked=(num_classes % CLASS_CHUNK != 0))
        lse = m_ref[...] + jnp.log(s_ref[...])
        row_loss = (-EPS_K * (t_ref[...] - num_classes * lse)
                    - (CONFIDENCE_K - EPS_K) * (g_ref[...] - lse))
        out_ref[...] = jnp.sum(row_loss).reshape(1, 1) / total_rows


def kernel(pred, target):
    rows, num_classes = pred.shape
    xt = pred.T
    num_chunks = pl.cdiv(num_classes, CLASS_CHUNK)
    tgt2d = target.astype(jnp.int32).reshape(1, rows)

    out = pl.pallas_call(
        functools.partial(_loss_kernel, num_chunks=num_chunks,
                          num_classes=num_classes, total_rows=rows),
        grid=(num_chunks,),
        in_specs=[
            pl.BlockSpec((CLASS_CHUNK, rows), lambda j: (j, 0)),
            pl.BlockSpec((1, rows), lambda j: (0, 0)),
        ],
        out_specs=pl.BlockSpec((1, 1), lambda j: (0, 0)),
        out_shape=jax.ShapeDtypeStruct((1, 1), jnp.float32),
        scratch_shapes=[
            pltpu.VMEM((1, rows), jnp.float32),
            pltpu.VMEM((1, rows), jnp.float32),
            pltpu.VMEM((1, rows), jnp.float32),
            pltpu.VMEM((1, rows), jnp.float32),
        ],
        compiler_params=pltpu.CompilerParams(
            vmem_limit_bytes=100 * 1024 * 1024),
    )(xt, tgt2d)
    return out[0, 0]
